# Initial kernel scaffold; baseline (speedup 1.0000x reference)
import jax
import jax.numpy as jnp
from jax.experimental import pallas as pl


def kernel(joint_ids, chain_emb_weight, depth_emb_weight, joint_to_chain, joint_to_depth):
    raise NotImplementedError("write your pallas kernel here")


# speedup vs baseline: 1.7679x; 1.7679x over previous
import functools

import jax
import jax.numpy as jnp
from jax import lax
from jax.experimental import pallas as pl
from jax.experimental.pallas import tpu as pltpu
from jax.experimental.pallas import tpu_sc as plsc

_NC = 2
_NS = 16
_NW = _NC * _NS
_L = 16

_D = 128
_HALF = 64
_NJ = 22
_CH = 256


def _sc_body(n_per_w, n_chunks, ids_hbm, chain_hbm, depth_hbm, jtc_hbm,
             jtd_hbm, out_hbm, jtc_v, jtd_v, ce_v, de_v, fused_v, ids_v,
             rows0, rows1, sem0, sem1, gsem):
    wid = lax.axis_index("s") * _NC + lax.axis_index("c")
    base = wid * n_per_w

    pltpu.sync_copy(jtc_hbm, jtc_v)
    pltpu.sync_copy(jtd_hbm, jtd_v)
    pltpu.async_copy(chain_hbm.at[jtc_v], ce_v, gsem).wait()
    pltpu.async_copy(depth_hbm.at[jtd_v], de_v, gsem).wait()
    for j in range(_NJ):
        for v in range(_HALF // _L):
            fused_v[pl.ds(j * _D + v * _L, _L)] = ce_v[j, pl.ds(v * _L, _L)]
            fused_v[pl.ds(j * _D + _HALF + v * _L, _L)] = (
                de_v[j, pl.ds(v * _L, _L)])

    pltpu.sync_copy(ids_hbm.at[pl.ds(base, n_per_w)], ids_v)
    lane_off = lax.iota(jnp.int32, _L) * _D

    slots = ((rows0, sem0), (rows1, sem1))

    def chunk_pair(k2, carry):
        for sl, (rows, sem) in enumerate(slots):
            k = k2 * 2 + sl

            @pl.when(k2 > 0)
            def _():
                pltpu.make_async_copy(
                    rows, out_hbm.at[pl.ds(0, _CH * _D)], sem).wait()

            def group(g, c2):
                off = pl.multiple_of(k * _CH + g * _L, _L)
                idv = ids_v[pl.ds(off, _L)]
                lb = idv * _D
                sb = lane_off + g * (_L * _D)
                for c in range(_D):
                    vals = plsc.load_gather(fused_v, [lb + c])
                    plsc.store_scatter(rows, [sb + c], vals)
                return c2

            lax.fori_loop(0, _CH // _L, group, 0)
            pltpu.async_copy(
                rows, out_hbm.at[pl.ds((base + k * _CH) * _D, _CH * _D)],
                sem)
        return carry

    lax.fori_loop(0, n_chunks // 2, chunk_pair, 0)
    for rows, sem in slots:
        pltpu.make_async_copy(rows, out_hbm.at[pl.ds(0, _CH * _D)],
                              sem).wait()


@functools.partial(jax.jit, static_argnames=())
def _sc_lookup(ids, chain_emb, depth_emb, jtc_pad, jtd_pad):
    n = ids.shape[0]
    assert n % (_NW * _CH) == 0
    n_per_w = n // _NW
    n_chunks = n_per_w // _CH
    assert n_chunks % 2 == 0

    mesh = plsc.VectorSubcoreMesh(core_axis_name="c", subcore_axis_name="s",
                                  num_cores=_NC, num_subcores=_NS)
    kern = pl.kernel(
        functools.partial(_sc_body, n_per_w, n_chunks),
        out_type=jax.ShapeDtypeStruct((n * _D,), jnp.float32),
        mesh=mesh,
        compiler_params=pltpu.CompilerParams(needs_layout_passes=False),
        scratch_types=[
            pltpu.VMEM((32,), jnp.int32),
            pltpu.VMEM((32,), jnp.int32),
            pltpu.VMEM((32, _D), jnp.float32),
            pltpu.VMEM((32, _D), jnp.float32),
            pltpu.VMEM((_NJ * _D,), jnp.float32),
            pltpu.VMEM((n_per_w,), jnp.int32),
            pltpu.VMEM((_CH * _D,), jnp.float32),
            pltpu.VMEM((_CH * _D,), jnp.float32),
            pltpu.SemaphoreType.DMA,
            pltpu.SemaphoreType.DMA,
            pltpu.SemaphoreType.DMA,
        ],
    )
    return kern(ids, chain_emb, depth_emb, jtc_pad, jtd_pad)


def kernel(joint_ids, chain_emb_weight, depth_emb_weight, joint_to_chain,
           joint_to_depth):
    b, t = joint_ids.shape
    ids = joint_ids.reshape(-1).astype(jnp.int32)
    jtc_pad = jnp.pad(joint_to_chain.astype(jnp.int32), (0, 10))
    jtd_pad = jnp.pad(joint_to_depth.astype(jnp.int32), (0, 10))
    ce_pad = jnp.pad(chain_emb_weight,
                     ((0, 8 - chain_emb_weight.shape[0]), (0, _D - _HALF)))
    de_pad = jnp.pad(depth_emb_weight,
                     ((0, 8 - depth_emb_weight.shape[0]), (0, _D - _HALF)))
    out = _sc_lookup(ids, ce_pad, de_pad, jtc_pad, jtd_pad)
    return out.reshape(b, t, _D)


# speedup vs baseline: 3.0943x; 1.7503x over previous
import functools

import jax
import jax.numpy as jnp
from jax import lax
from jax.experimental import pallas as pl
from jax.experimental.pallas import tpu as pltpu
from jax.experimental.pallas import tpu_sc as plsc

_NC = 2
_NS = 16
_NW = _NC * _NS
_L = 16

_D = 128
_HALF = 64
_NJ = 32
_CH = 128


def _mesh():
    return plsc.VectorSubcoreMesh(core_axis_name="c", subcore_axis_name="s",
                                  num_cores=_NC, num_subcores=_NS)


def _build_body(chain_hbm, depth_hbm, jtc_hbm, jtd_hbm, fused_hbm,
                jtc_v, jtd_v, ce_v, de_v, fused_v, gsem):
    wid = lax.axis_index("s") * _NC + lax.axis_index("c")

    @pl.when(wid == 0)
    def _():
        pltpu.sync_copy(jtc_hbm, jtc_v)
        pltpu.sync_copy(jtd_hbm, jtd_v)
        pltpu.async_copy(chain_hbm.at[jtc_v], ce_v, gsem).wait()
        pltpu.async_copy(depth_hbm.at[jtd_v], de_v, gsem).wait()
        for j in range(_NJ):
            for v in range(_HALF // _L):
                fused_v[j, pl.ds(v * _L, _L)] = ce_v[j, pl.ds(v * _L, _L)]
                fused_v[j, pl.ds(_HALF + v * _L, _L)] = (
                    de_v[j, pl.ds(v * _L, _L)])
        pltpu.sync_copy(fused_v, fused_hbm)


def _lookup_body(rows_per_w, ids_hbm, fused_hbm, out_hbm, ids_v,
                 rows0, rows1, gsem0, gsem1, ssem0, ssem1):
    wid = lax.axis_index("s") * _NC + lax.axis_index("c")
    rbase = pl.multiple_of(wid * rows_per_w, 8)
    pltpu.sync_copy(ids_hbm.at[pl.ds(rbase, rows_per_w), :], ids_v)

    slots = ((rows0, gsem0, ssem0), (rows1, gsem1, ssem1))

    def chunk_pair(k2, carry):
        for sl, (rows, gsem, ssem) in enumerate(slots):
            k = k2 * 2 + sl

            @pl.when(k2 > 0)
            def _():
                pltpu.make_async_copy(
                    rows, out_hbm.at[pl.ds(0, _CH), :], ssem).wait()

            pltpu.async_copy(fused_hbm.at[ids_v.at[k]], rows, gsem).wait()
            off = pl.multiple_of((rbase + k) * _CH, _CH)
            pltpu.async_copy(rows, out_hbm.at[pl.ds(off, _CH), :], ssem)
        return carry

    lax.fori_loop(0, rows_per_w // 2, chunk_pair, 0)
    for rows, _, ssem in slots:
        pltpu.make_async_copy(rows, out_hbm.at[pl.ds(0, _CH), :], ssem).wait()


@jax.jit
def _sc_encode(ids2d, chain_pad, depth_pad, jtc_pad, jtd_pad):
    n_rows = ids2d.shape[0]
    assert n_rows % (_NW * 2) == 0
    rows_per_w = n_rows // _NW

    build = pl.kernel(
        _build_body,
        out_type=jax.ShapeDtypeStruct((_NJ, _D), jnp.float32),
        mesh=_mesh(),
        compiler_params=pltpu.CompilerParams(needs_layout_passes=False),
        scratch_types=[
            pltpu.VMEM((_NJ,), jnp.int32),
            pltpu.VMEM((_NJ,), jnp.int32),
            pltpu.VMEM((_NJ, _D), jnp.float32),
            pltpu.VMEM((_NJ, _D), jnp.float32),
            pltpu.VMEM((_NJ, _D), jnp.float32),
            pltpu.SemaphoreType.DMA,
        ],
    )
    fused = build(chain_pad, depth_pad, jtc_pad, jtd_pad)

    lookup = pl.kernel(
        functools.partial(_lookup_body, rows_per_w),
        out_type=jax.ShapeDtypeStruct((n_rows * _CH, _D), jnp.float32),
        mesh=_mesh(),
        compiler_params=pltpu.CompilerParams(needs_layout_passes=False),
        scratch_types=[
            pltpu.VMEM((rows_per_w, _CH), jnp.int32),
            pltpu.VMEM((_CH, _D), jnp.float32),
            pltpu.VMEM((_CH, _D), jnp.float32),
            pltpu.SemaphoreType.DMA,
            pltpu.SemaphoreType.DMA,
            pltpu.SemaphoreType.DMA,
            pltpu.SemaphoreType.DMA,
        ],
    )
    return lookup(ids2d, fused)


def kernel(joint_ids, chain_emb_weight, depth_emb_weight, joint_to_chain,
           joint_to_depth):
    b, t = joint_ids.shape
    ids2d = joint_ids.reshape(-1, _CH).astype(jnp.int32)
    jtc_pad = jnp.pad(joint_to_chain.astype(jnp.int32), (0, 10))
    jtd_pad = jnp.pad(joint_to_depth.astype(jnp.int32), (0, 10))
    ce_pad = jnp.pad(chain_emb_weight,
                     ((0, 8 - chain_emb_weight.shape[0]), (0, _D - _HALF)))
    de_pad = jnp.pad(depth_emb_weight,
                     ((0, 8 - depth_emb_weight.shape[0]), (0, _D - _HALF)))
    out = _sc_encode(ids2d, ce_pad, de_pad, jtc_pad, jtd_pad)
    return out.reshape(b, t, _D)


# speedup vs baseline: 32.1149x; 10.3787x over previous
import functools

import jax
import jax.numpy as jnp
from jax import lax
from jax.experimental import pallas as pl
from jax.experimental.pallas import tpu as pltpu
from jax.experimental.pallas import tpu_sc as plsc

_NC = 2
_NS = 16
_NW = _NC * _NS
_L = 16

_D = 128
_HALF = 64
_NJ = 32
_CH = 128


def _body(rows_per_w, ids_hbm, chain_hbm, depth_hbm, jtc_hbm, jtd_hbm,
          out_hbm, jtc_v, jtd_v, ce_v, de_v, fused_v, fused_sh, ids_v,
          rows0, rows1, gsem0, gsem1, ssem0, ssem1):
    cid = lax.axis_index("c")
    sid = lax.axis_index("s")
    wid = sid * _NC + cid

    @pl.when(sid == 0)
    def _():
        pltpu.sync_copy(jtc_hbm, jtc_v)
        pltpu.sync_copy(jtd_hbm, jtd_v)
        pltpu.async_copy(chain_hbm.at[jtc_v], ce_v, gsem0).wait()
        pltpu.async_copy(depth_hbm.at[jtd_v], de_v, gsem0).wait()
        for j in range(_NJ):
            for v in range(_HALF // _L):
                fused_v[j, pl.ds(v * _L, _L)] = ce_v[j, pl.ds(v * _L, _L)]
                fused_v[j, pl.ds(_HALF + v * _L, _L)] = (
                    de_v[j, pl.ds(v * _L, _L)])
        pltpu.sync_copy(fused_v, fused_sh)

    rbase = pl.multiple_of(wid * rows_per_w, 8)
    pltpu.async_copy(ids_hbm.at[pl.ds(rbase, rows_per_w), :], ids_v,
                     gsem1).wait()
    plsc.subcore_barrier()

    slots = ((rows0, gsem0, ssem0), (rows1, gsem1, ssem1))

    def chunk_pair(k2, carry):
        for sl, (rows, gsem, ssem) in enumerate(slots):
            k = k2 * 2 + sl

            @pl.when(k2 > 0)
            def _():
                pltpu.make_async_copy(
                    rows, out_hbm.at[pl.ds(0, _CH), :], ssem).wait()

            pltpu.async_copy(fused_sh.at[ids_v.at[k]], rows, gsem).wait()
            off = pl.multiple_of((rbase + k) * _CH, _CH)
            pltpu.async_copy(rows, out_hbm.at[pl.ds(off, _CH), :], ssem)
        return carry

    lax.fori_loop(0, rows_per_w // 2, chunk_pair, 0)
    for rows, _, ssem in slots:
        pltpu.make_async_copy(rows, out_hbm.at[pl.ds(0, _CH), :], ssem).wait()


@jax.jit
def _sc_encode(ids2d, chain_pad, depth_pad, jtc_pad, jtd_pad):
    n_rows = ids2d.shape[0]
    assert n_rows % (_NW * 2) == 0
    rows_per_w = n_rows // _NW

    mesh = plsc.VectorSubcoreMesh(core_axis_name="c", subcore_axis_name="s",
                                  num_cores=_NC, num_subcores=_NS)
    lookup = pl.kernel(
        functools.partial(_body, rows_per_w),
        out_type=jax.ShapeDtypeStruct((n_rows * _CH, _D), jnp.float32),
        mesh=mesh,
        compiler_params=pltpu.CompilerParams(needs_layout_passes=False),
        scratch_types=[
            pltpu.VMEM((_NJ,), jnp.int32),
            pltpu.VMEM((_NJ,), jnp.int32),
            pltpu.VMEM((_NJ, _D), jnp.float32),
            pltpu.VMEM((_NJ, _D), jnp.float32),
            pltpu.VMEM((_NJ, _D), jnp.float32),
            pltpu.VMEM_SHARED((_NJ, _D), jnp.float32),
            pltpu.VMEM((rows_per_w, _CH), jnp.int32),
            pltpu.VMEM((_CH, _D), jnp.float32),
            pltpu.VMEM((_CH, _D), jnp.float32),
            pltpu.SemaphoreType.DMA,
            pltpu.SemaphoreType.DMA,
            pltpu.SemaphoreType.DMA,
            pltpu.SemaphoreType.DMA,
        ],
    )
    return lookup(ids2d, chain_pad, depth_pad, jtc_pad, jtd_pad)


def kernel(joint_ids, chain_emb_weight, depth_emb_weight, joint_to_chain,
           joint_to_depth):
    b, t = joint_ids.shape
    ids2d = joint_ids.reshape(-1, _CH).astype(jnp.int32)
    jtc_pad = jnp.pad(joint_to_chain.astype(jnp.int32), (0, 10))
    jtd_pad = jnp.pad(joint_to_depth.astype(jnp.int32), (0, 10))
    ce_pad = jnp.pad(chain_emb_weight,
                     ((0, 8 - chain_emb_weight.shape[0]), (0, _D - _HALF)))
    de_pad = jnp.pad(depth_emb_weight,
                     ((0, 8 - depth_emb_weight.shape[0]), (0, _D - _HALF)))
    out = _sc_encode(ids2d, ce_pad, de_pad, jtc_pad, jtd_pad)
    return out.reshape(b, t, _D)


# speedup vs baseline: 33.3163x; 1.0374x over previous
import functools

import jax
import jax.numpy as jnp
from jax import lax
from jax.experimental import pallas as pl
from jax.experimental.pallas import tpu as pltpu
from jax.experimental.pallas import tpu_sc as plsc

_NC = 2
_NS = 16
_NW = _NC * _NS
_L = 16

_D = 128
_HALF = 64
_NJ = 32
_CH = 128


def _body(rows_per_w, ids_hbm, chain_hbm, depth_hbm, jtc_hbm, jtd_hbm,
          out_hbm, jtc_v, jtd_v, ce_v, de_v, fused_v, fused_sh, ids_v,
          rows0, rows1, gsem0, gsem1, ssem0, ssem1):
    cid = lax.axis_index("c")
    sid = lax.axis_index("s")
    wid = sid * _NC + cid

    @pl.when(sid == 0)
    def _():
        pltpu.sync_copy(jtc_hbm, jtc_v)
        pltpu.sync_copy(jtd_hbm, jtd_v)
        pltpu.async_copy(chain_hbm.at[jtc_v], ce_v, gsem0).wait()
        pltpu.async_copy(depth_hbm.at[jtd_v], de_v, gsem0).wait()
        for j in range(_NJ):
            for v in range(_HALF // _L):
                fused_v[j, pl.ds(v * _L, _L)] = ce_v[j, pl.ds(v * _L, _L)]
                fused_v[j, pl.ds(_HALF + v * _L, _L)] = (
                    de_v[j, pl.ds(v * _L, _L)])
        pltpu.sync_copy(fused_v, fused_sh)

    rbase = pl.multiple_of(wid * rows_per_w, 8)
    pltpu.async_copy(ids_hbm.at[pl.ds(rbase, rows_per_w), :], ids_v,
                     gsem1).wait()
    plsc.subcore_barrier()

    slots = ((rows0, gsem0, ssem0), (rows1, gsem1, ssem1))

    def chunk_pair(k2, carry):
        for sl, (rows, gsem, ssem) in enumerate(slots):
            k = (k2 * 2 + sl) * 2

            @pl.when(k2 > 0)
            def _():
                pltpu.make_async_copy(
                    rows, out_hbm.at[pl.ds(0, 2 * _CH), :], ssem).wait()

            cp0 = pltpu.async_copy(
                fused_sh.at[ids_v.at[k]], rows.at[pl.ds(0, _CH), :], gsem)
            cp1 = pltpu.async_copy(
                fused_sh.at[ids_v.at[k + 1]], rows.at[pl.ds(_CH, _CH), :],
                gsem)
            cp0.wait()
            cp1.wait()
            off = pl.multiple_of((rbase + k) * _CH, _CH)
            pltpu.async_copy(rows, out_hbm.at[pl.ds(off, 2 * _CH), :], ssem)
        return carry

    lax.fori_loop(0, rows_per_w // 4, chunk_pair, 0)
    for rows, _, ssem in slots:
        pltpu.make_async_copy(rows, out_hbm.at[pl.ds(0, 2 * _CH), :],
                              ssem).wait()


@jax.jit
def _sc_encode(ids2d, chain_pad, depth_pad, jtc_pad, jtd_pad):
    n_rows = ids2d.shape[0]
    assert n_rows % (_NW * 4) == 0
    rows_per_w = n_rows // _NW

    mesh = plsc.VectorSubcoreMesh(core_axis_name="c", subcore_axis_name="s",
                                  num_cores=_NC, num_subcores=_NS)
    lookup = pl.kernel(
        functools.partial(_body, rows_per_w),
        out_type=jax.ShapeDtypeStruct((n_rows * _CH, _D), jnp.float32),
        mesh=mesh,
        compiler_params=pltpu.CompilerParams(needs_layout_passes=False),
        scratch_types=[
            pltpu.VMEM((_NJ,), jnp.int32),
            pltpu.VMEM((_NJ,), jnp.int32),
            pltpu.VMEM((_NJ, _D), jnp.float32),
            pltpu.VMEM((_NJ, _D), jnp.float32),
            pltpu.VMEM((_NJ, _D), jnp.float32),
            pltpu.VMEM_SHARED((_NJ, _D), jnp.float32),
            pltpu.VMEM((rows_per_w, _CH), jnp.int32),
            pltpu.VMEM((2 * _CH, _D), jnp.float32),
            pltpu.VMEM((2 * _CH, _D), jnp.float32),
            pltpu.SemaphoreType.DMA,
            pltpu.SemaphoreType.DMA,
            pltpu.SemaphoreType.DMA,
            pltpu.SemaphoreType.DMA,
        ],
    )
    return lookup(ids2d, chain_pad, depth_pad, jtc_pad, jtd_pad)


def kernel(joint_ids, chain_emb_weight, depth_emb_weight, joint_to_chain,
           joint_to_depth):
    b, t = joint_ids.shape
    ids2d = joint_ids.reshape(-1, _CH).astype(jnp.int32)
    jtc_pad = jnp.pad(joint_to_chain.astype(jnp.int32), (0, 10))
    jtd_pad = jnp.pad(joint_to_depth.astype(jnp.int32), (0, 10))
    ce_pad = jnp.pad(chain_emb_weight,
                     ((0, 8 - chain_emb_weight.shape[0]), (0, _D - _HALF)))
    de_pad = jnp.pad(depth_emb_weight,
                     ((0, 8 - depth_emb_weight.shape[0]), (0, _D - _HALF)))
    out = _sc_encode(ids2d, ce_pad, de_pad, jtc_pad, jtd_pad)
    return out.reshape(b, t, _D)
